# Initial kernel scaffold; baseline (speedup 1.0000x reference)
import jax
import jax.numpy as jnp
from jax.experimental import pallas as pl


def kernel(x, edge_index, edge_type, W_root0, W_rel0, W_root1, W_rel1):
    raise NotImplementedError("write your pallas kernel here")


# speedup vs baseline: 17.3349x; 17.3349x over previous
import functools

import jax
import jax.numpy as jnp
from jax import lax
from jax.experimental import pallas as pl
from jax.experimental.pallas import tpu as pltpu
from jax.experimental.pallas import tpu_sc as plsc

N = 10000
E = 320000
D = 128
H = 128
R = 8

NC = 2
NS = 16
NW = NC * NS
C = 80
ROWS = E // C
RPT = ROWS // NW
RPC = ROWS // NS
NR = N * R
CPT = NR // NS
NPAD = 10240
APT = NPAD // NS

_mesh = plsc.VectorSubcoreMesh(core_axis_name="c", subcore_axis_name="s")


@functools.partial(
    pl.kernel,
    out_type=(
        jax.ShapeDtypeStruct((NW, RPT, C), jnp.int32),
        jax.ShapeDtypeStruct((NW, RPT, C), jnp.float32),
    ),
    mesh=_mesh,
    compiler_params=pltpu.CompilerParams(use_tc_tiling_on_sc=False, needs_layout_passes=False),
    scratch_types=[
        pltpu.VMEM_SHARED((NR,), jnp.float32),
        pltpu.VMEM((RPC, C), jnp.int32),
        pltpu.VMEM((RPC, C), jnp.int32),
        pltpu.VMEM((RPC, C), jnp.int32),
        pltpu.VMEM((RPT, C), jnp.int32),
        pltpu.VMEM((RPT, C), jnp.float32),
        pltpu.VMEM((C,), jnp.float32),
        pltpu.VMEM((C,), jnp.float32),
        pltpu.VMEM((CPT + 8,), jnp.float32),
    ],
)
def _prep(src3, dst3, et3, srckey_out, norm_out,
          cnt, dstb, etb, keyb, srcb, nrmb, cvals, ones, zbuf):
    c = lax.axis_index("c")
    s = lax.axis_index("s")
    wid = c * NS + s

    def zrow(i, carry):
        zbuf[pl.ds(i * 16, 16)] = jnp.zeros((16,), jnp.float32)
        return carry

    lax.fori_loop(0, (CPT + 8) // 16, zrow, 0)
    pltpu.sync_copy(zbuf.at[pl.ds(0, CPT)], cnt.at[pl.ds(s * CPT, CPT)])
    for v in range(C // 16):
        ones[pl.ds(v * 16, 16)] = jnp.full((16,), 1.0, jnp.float32)
    plsc.subcore_barrier()

    pltpu.sync_copy(dst3.at[2 * s], dstb.at[pl.ds(0, RPT)])
    pltpu.sync_copy(dst3.at[2 * s + 1], dstb.at[pl.ds(RPT, RPT)])
    pltpu.sync_copy(et3.at[2 * s], etb.at[pl.ds(0, RPT)])
    pltpu.sync_copy(et3.at[2 * s + 1], etb.at[pl.ds(RPT, RPT)])

    def keyrow(j, carry):
        for v in range(C // 16):
            sl = pl.ds(v * 16, 16)
            keyb[j, sl] = dstb[j, sl] * R + etb[j, sl]
        return carry

    lax.fori_loop(0, RPC, keyrow, 0)

    def cntrow(j, carry):
        pltpu.sync_copy(ones, cnt.at[keyb.at[j]], add=True)
        return carry

    lax.fori_loop(0, RPC, cntrow, 0)
    plsc.subcore_barrier()

    pltpu.sync_copy(src3.at[wid], srcb)
    pltpu.sync_copy(dst3.at[wid], dstb.at[pl.ds(0, RPT)])
    pltpu.sync_copy(et3.at[wid], etb.at[pl.ds(0, RPT)])

    def nrow(j, carry):
        for v in range(C // 16):
            sl = pl.ds(v * 16, 16)
            keyb[j, sl] = dstb[j, sl] * R + etb[j, sl]
            srcb[j, sl] = srcb[j, sl] * R + etb[j, sl]
        pltpu.sync_copy(cnt.at[keyb.at[j]], cvals)
        for v in range(C // 16):
            sl = pl.ds(v * 16, 16)
            nrmb[j, sl] = 1.0 / jnp.maximum(cvals[sl], 1.0)
        return carry

    lax.fori_loop(0, RPT, nrow, 0)
    pltpu.sync_copy(srcb, srckey_out.at[wid])
    pltpu.sync_copy(nrmb, norm_out.at[wid])


@functools.partial(
    pl.kernel,
    out_type=jax.ShapeDtypeStruct((NC, NPAD, H), jnp.float32),
    mesh=_mesh,
    compiler_params=pltpu.CompilerParams(use_tc_tiling_on_sc=False, needs_layout_passes=False),
    scratch_types=[
        pltpu.VMEM_SHARED((NPAD, H), jnp.float32),
        pltpu.VMEM((RPT, C), jnp.int32),
        pltpu.VMEM((RPT, C), jnp.int32),
        pltpu.VMEM((RPT, C), jnp.float32),
        pltpu.VMEM((C, H), jnp.float32),
        pltpu.SemaphoreType.DMA,
    ],
)
def _agg(hr, srckey3, dst3, norm3, parts,
         acc, skb, dkb, nrmb, rows, sem):
    c = lax.axis_index("c")
    s = lax.axis_index("s")
    wid = c * NS + s

    def zrow(i, carry):
        for k in range(H // 16):
            rows[i, pl.ds(k * 16, 16)] = jnp.zeros((16,), jnp.float32)
        return carry

    lax.fori_loop(0, C, zrow, 0)
    for t in range(APT // C):
        pltpu.sync_copy(rows, acc.at[pl.ds(s * APT + t * C, C)])
    pltpu.sync_copy(srckey3.at[wid], skb)
    pltpu.sync_copy(dst3.at[wid], dkb)
    pltpu.sync_copy(norm3.at[wid], nrmb)
    plsc.subcore_barrier()

    def edge_chunk(j, carry):
        pltpu.async_copy(hr.at[skb.at[j]], rows, sem).wait()

        def scale(e, carry2):
            nv = plsc.load_gather(nrmb, [jnp.full((16,), j, jnp.int32),
                                         jnp.full((16,), e, jnp.int32)])
            for k in range(H // 16):
                sl = pl.ds(k * 16, 16)
                rows[e, sl] = rows[e, sl] * nv
            return carry2

        lax.fori_loop(0, C, scale, 0)
        pltpu.sync_copy(rows, acc.at[dkb.at[j]], add=True)
        return carry

    lax.fori_loop(0, RPT, edge_chunk, 0)
    plsc.subcore_barrier()
    for t in range(APT // C):
        pltpu.sync_copy(acc.at[pl.ds(s * APT + t * C, C)], rows)
        pltpu.sync_copy(rows, parts.at[c, pl.ds(s * APT + t * C, C)])


def _mm_body(x_ref, w_ref, o_ref):
    o_ref[...] = jnp.dot(x_ref[...], w_ref[...],
                         preferred_element_type=jnp.float32)


def _matmul(x, w, bn=1000):
    n, k = x.shape
    m = w.shape[1]
    return pl.pallas_call(
        _mm_body,
        grid=(n // bn,),
        in_specs=[pl.BlockSpec((bn, k), lambda i: (i, 0)),
                  pl.BlockSpec((k, m), lambda i: (0, 0))],
        out_specs=pl.BlockSpec((bn, m), lambda i: (i, 0)),
        out_shape=jax.ShapeDtypeStruct((n, m), jnp.float32),
    )(x, w)


def _comb_body(mode, x_ref, w_ref, p_ref, o_ref):
    h = jnp.dot(x_ref[...], w_ref[...], preferred_element_type=jnp.float32)
    h = h + p_ref[0] + p_ref[1]
    if mode == "relu":
        h = jnp.maximum(h, 0.0)
    else:
        nrm = jnp.sqrt(jnp.sum(h * h, axis=-1, keepdims=True))
        h = h / jnp.maximum(nrm, 1e-12)
    o_ref[...] = h


def _combine(mode, x, w, parts, bn=1000):
    n, k = x.shape
    m = w.shape[1]
    return pl.pallas_call(
        functools.partial(_comb_body, mode),
        grid=(n // bn,),
        in_specs=[pl.BlockSpec((bn, k), lambda i: (i, 0)),
                  pl.BlockSpec((k, m), lambda i: (0, 0)),
                  pl.BlockSpec((NC, bn, m), lambda i: (0, i, 0))],
        out_specs=pl.BlockSpec((bn, m), lambda i: (i, 0)),
        out_shape=jax.ShapeDtypeStruct((n, m), jnp.float32),
    )(x, w, parts)


def kernel(x, edge_index, edge_type, W_root0, W_rel0, W_root1, W_rel1):
    src = edge_index[0]
    dst = edge_index[1]
    src3 = src.reshape(NW, RPT, C)
    dst3 = dst.reshape(NW, RPT, C)
    et3 = edge_type.reshape(NW, RPT, C)
    srckey3, norm3 = _prep(src3, dst3, et3)

    Wf0 = W_rel0.transpose(1, 0, 2).reshape(D, R * H)
    Wf1 = W_rel1.transpose(1, 0, 2).reshape(H, R * H)

    hr = _matmul(x, Wf0).reshape(NR, H)
    parts = _agg(hr, srckey3, dst3, norm3)
    h = _combine("relu", x, W_root0, parts)

    hr = _matmul(h, Wf1).reshape(NR, H)
    parts = _agg(hr, srckey3, dst3, norm3)
    return _combine("l2", h, W_root1, parts)
